# baseline (speedup 1.0000x reference)
import functools

import jax
import jax.numpy as jnp
from jax import lax
from jax.experimental import pallas as pl
from jax.experimental.pallas import tpu as pltpu
from jax.experimental.pallas import tpu_sc as plsc


def kernel(idx, mu_weight, sigma_weight):
    B = idx.shape[0]
    V, D = mu_weight.shape
    info = plsc.get_sparse_core_info()
    NC, NS, L = info.num_cores, info.num_subcores, info.num_lanes
    NW = NC * NS
    assert B % (L * NW) == 0 and D % L == 0 and V % 2 == 0
    bpw = B // NW

    one = lax.optimization_barrier(jnp.float32(1.0))
    mu2 = (mu_weight * one).reshape(V // 2, 2 * D)
    sig2 = (sigma_weight * one).reshape(V // 2, 2 * D)

    mesh = plsc.VectorSubcoreMesh(core_axis_name="c", subcore_axis_name="s")

    @functools.partial(
        pl.kernel,
        mesh=mesh,
        compiler_params=pltpu.CompilerParams(use_tc_tiling_on_sc=True,
                                             skip_device_barrier=True),
        out_type=jax.ShapeDtypeStruct((2 * B, D), jnp.float32),
        scratch_types=[
            pltpu.VMEM((bpw,), jnp.int32),
            pltpu.VMEM((bpw,), jnp.int32),
            pltpu.VMEM((bpw, 2 * D), jnp.float32),
            pltpu.VMEM((bpw, 2 * D), jnp.float32),
            pltpu.VMEM((2 * bpw, D), jnp.float32),
            pltpu.SemaphoreType.DMA,
            pltpu.SemaphoreType.DMA,
        ],
    )
    def run(idx_hbm, mu_hbm, sig_hbm, out_hbm,
            idx_v, pair_v, mu_v, sig_v, out_v, sem_mu, sem_sig):
        wid = lax.axis_index("s") * NC + lax.axis_index("c")
        base = wid * bpw
        pltpu.sync_copy(idx_hbm.at[pl.ds(base, bpw)], idx_v)

        for i in range(bpw // L):
            rv = idx_v[pl.ds(i * L, L)]
            pair_v[pl.ds(i * L, L)] = rv >> 1

        mu_cp = pltpu.async_copy(mu_hbm.at[pair_v], mu_v, sem_mu)
        sig_cp = pltpu.async_copy(sig_hbm.at[pair_v], sig_v, sem_sig)
        mu_cp.wait()
        sig_cp.wait()

        for i in range(bpw // L):
            rv = idx_v[pl.ds(i * L, L)]
            for l in range(L):
                j = i * L + l
                off = (rv[l] & 1) * D
                for cb in range(D // L):
                    mv = mu_v[j, pl.ds(off + cb * L, L)]
                    out_v[2 * j, pl.ds(cb * L, L)] = mv
                for cb in range(D // L):
                    sv = sig_v[j, pl.ds(off + cb * L, L)]
                    out_v[2 * j + 1, pl.ds(cb * L, L)] = (
                        jnp.maximum(sv, 0.0) + jnp.exp(jnp.minimum(sv, 0.0)))

        pltpu.sync_copy(out_v, out_hbm.at[pl.ds(2 * base, 2 * bpw)])

    out2 = run(idx, mu2, sig2)
    return out2.reshape(B, 2 * D)


# speedup vs baseline: 1.1932x; 1.1932x over previous
import functools

import jax
import jax.numpy as jnp
from jax import lax
from jax.experimental import pallas as pl
from jax.experimental.pallas import tpu as pltpu
from jax.experimental.pallas import tpu_sc as plsc


def kernel(idx, mu_weight, sigma_weight):
    B = idx.shape[0]
    V, D = mu_weight.shape
    info = plsc.get_sparse_core_info()
    NC, NS, L = info.num_cores, info.num_subcores, info.num_lanes
    NW = NC * NS
    assert B % (L * NW) == 0 and D % L == 0 and V % 2 == 0
    bpw = B // NW

    both2 = jnp.concatenate([mu_weight.reshape(V // 2, 2 * D),
                             sigma_weight.reshape(V // 2, 2 * D)], axis=1)

    mesh = plsc.VectorSubcoreMesh(core_axis_name="c", subcore_axis_name="s")

    @functools.partial(
        pl.kernel,
        mesh=mesh,
        compiler_params=pltpu.CompilerParams(use_tc_tiling_on_sc=True,
                                             skip_device_barrier=True),
        out_type=jax.ShapeDtypeStruct((2 * B, D), jnp.float32),
        scratch_types=[
            pltpu.VMEM((bpw,), jnp.int32),
            pltpu.VMEM((bpw,), jnp.int32),
            pltpu.VMEM((bpw, 4 * D), jnp.float32),
            pltpu.VMEM((2 * bpw, D), jnp.float32),
            pltpu.SemaphoreType.DMA,
        ],
    )
    def run(idx_hbm, both_hbm, out_hbm,
            idx_v, pair_v, g_v, out_v, sem_g):
        wid = lax.axis_index("s") * NC + lax.axis_index("c")
        base = wid * bpw
        pltpu.sync_copy(idx_hbm.at[pl.ds(base, bpw)], idx_v)

        for i in range(bpw // L):
            rv = idx_v[pl.ds(i * L, L)]
            pair_v[pl.ds(i * L, L)] = rv >> 1

        pltpu.async_copy(both_hbm.at[pair_v], g_v, sem_g).wait()

        for i in range(bpw // L):
            rv = idx_v[pl.ds(i * L, L)]
            for l in range(L):
                j = i * L + l
                off = (rv[l] & 1) * D
                for cb in range(D // L):
                    mv = g_v[j, pl.ds(off + cb * L, L)]
                    out_v[2 * j, pl.ds(cb * L, L)] = mv
                for cb in range(D // L):
                    sv = g_v[j, pl.ds(2 * D + off + cb * L, L)]
                    out_v[2 * j + 1, pl.ds(cb * L, L)] = (
                        jnp.maximum(sv, 0.0) + jnp.exp(jnp.minimum(sv, 0.0)))

        pltpu.sync_copy(out_v, out_hbm.at[pl.ds(2 * base, 2 * bpw)])

    out2 = run(idx, both2)
    return out2.reshape(B, 2 * D)


# speedup vs baseline: 1.6078x; 1.3474x over previous
import functools

import jax
import jax.numpy as jnp
from jax import lax
from jax.experimental import pallas as pl
from jax.experimental.pallas import tpu as pltpu
from jax.experimental.pallas import tpu_sc as plsc


def _gather_kernel(B, D, NC, L, bpw, act):
    mesh = plsc.VectorSubcoreMesh(core_axis_name="c", subcore_axis_name="s")

    @functools.partial(
        pl.kernel,
        mesh=mesh,
        compiler_params=pltpu.CompilerParams(use_tc_tiling_on_sc=False),
        out_type=jax.ShapeDtypeStruct((B, D), jnp.float32),
        scratch_types=[
            pltpu.VMEM((bpw,), jnp.int32),
            pltpu.VMEM((bpw, D), jnp.float32),
            pltpu.SemaphoreType.DMA,
        ],
    )
    def run(idx_hbm, tbl_hbm, out_hbm, idx_v, rows_v, sem):
        wid = lax.axis_index("s") * NC + lax.axis_index("c")
        base = wid * bpw
        pltpu.sync_copy(idx_hbm.at[pl.ds(base, bpw)], idx_v)
        pltpu.async_copy(tbl_hbm.at[idx_v], rows_v, sem).wait()
        if act:
            rows_per_iter = 4

            def body(i, carry):
                r0 = i * rows_per_iter
                for rr in range(rows_per_iter):
                    for j in range(D // L):
                        x = rows_v[r0 + rr, pl.ds(j * L, L)]
                        rows_v[r0 + rr, pl.ds(j * L, L)] = (
                            jnp.maximum(x, 0.0) + jnp.exp(jnp.minimum(x, 0.0)))
                return carry

            lax.fori_loop(0, bpw // rows_per_iter, body, 0)
        pltpu.sync_copy(rows_v, out_hbm.at[pl.ds(base, bpw)])

    return run


def kernel(idx, mu_weight, sigma_weight):
    B = idx.shape[0]
    V, D = mu_weight.shape
    info = plsc.get_sparse_core_info()
    NC, NS, L = info.num_cores, info.num_subcores, info.num_lanes
    NW = NC * NS
    assert B % NW == 0 and D % L == 0
    bpw = B // NW

    mu_run = _gather_kernel(B, D, NC, L, bpw, act=False)
    sig_run = _gather_kernel(B, D, NC, L, bpw, act=True)
    mu_emb = mu_run(idx, mu_weight)
    sig_act = sig_run(idx, sigma_weight)
    return jnp.concatenate([mu_emb, sig_act], axis=1)


# speedup vs baseline: 1.6488x; 1.0255x over previous
import functools

import jax
import jax.numpy as jnp
from jax import lax
from jax.experimental import pallas as pl
from jax.experimental.pallas import tpu as pltpu
from jax.experimental.pallas import tpu_sc as plsc


def kernel(idx, mu_weight, sigma_weight):
    B = idx.shape[0]
    V, D = mu_weight.shape
    info = plsc.get_sparse_core_info()
    NC, NS, L = info.num_cores, info.num_subcores, info.num_lanes
    NW = NC * NS
    assert B % NW == 0 and D % L == 0
    bpw = B // NW

    mesh = plsc.VectorSubcoreMesh(core_axis_name="c", subcore_axis_name="s")

    @functools.partial(
        pl.kernel,
        mesh=mesh,
        compiler_params=pltpu.CompilerParams(use_tc_tiling_on_sc=False),
        out_type=jax.ShapeDtypeStruct((2 * B, D), jnp.float32),
        scratch_types=[
            pltpu.VMEM((bpw,), jnp.int32),
            pltpu.VMEM((bpw,), jnp.int32),
            pltpu.VMEM((bpw,), jnp.int32),
            pltpu.VMEM((bpw, D), jnp.float32),
            pltpu.VMEM((bpw, D), jnp.float32),
            pltpu.SemaphoreType.DMA,
            pltpu.SemaphoreType.DMA,
            pltpu.SemaphoreType.DMA,
            pltpu.SemaphoreType.DMA,
        ],
    )
    def run(idx_hbm, mu_hbm, sig_hbm, out_hbm,
            idx_v, evn_v, odd_v, mu_v, sig_v,
            sem_mu, sem_sig, sem_omu, sem_osig):
        wid = lax.axis_index("s") * NC + lax.axis_index("c")
        base = wid * bpw
        pltpu.sync_copy(idx_hbm.at[pl.ds(base, bpw)], idx_v)
        mu_cp = pltpu.async_copy(mu_hbm.at[idx_v], mu_v, sem_mu)
        sig_cp = pltpu.async_copy(sig_hbm.at[idx_v], sig_v, sem_sig)

        lane = lax.iota(jnp.int32, L)
        for j in range(bpw // L):
            evn = (base + j * L + lane) * 2
            evn_v[pl.ds(j * L, L)] = evn
            odd_v[pl.ds(j * L, L)] = evn + 1

        mu_cp.wait()
        omu_cp = pltpu.async_copy(mu_v, out_hbm.at[evn_v], sem_omu)

        sig_cp.wait()
        rows_per_iter = 4

        def body(i, carry):
            r0 = i * rows_per_iter
            for rr in range(rows_per_iter):
                for j in range(D // L):
                    x = sig_v[r0 + rr, pl.ds(j * L, L)]
                    sig_v[r0 + rr, pl.ds(j * L, L)] = (
                        jnp.maximum(x, 0.0) + jnp.exp(jnp.minimum(x, 0.0)))
            return carry

        lax.fori_loop(0, bpw // rows_per_iter, body, 0)

        osig_cp = pltpu.async_copy(sig_v, out_hbm.at[odd_v], sem_osig)
        omu_cp.wait()
        osig_cp.wait()

    out2 = run(idx, mu_weight, sigma_weight)
    return out2.reshape(B, 2 * D)
